# baseline (device time: 25212 ns/iter reference)
import jax
import jax.numpy as jnp
from jax import lax
from jax.experimental import pallas as pl
from jax.experimental.pallas import tpu as pltpu

N_DEV = 4


def kernel(x, Win0, Wout0, Win1, Wout1, Win2, Wout2):
    b_rows, d = x.shape
    rb = b_rows // N_DEV

    def body(x_ref, win0_ref, wout0_ref, win1_any, wout1_any, win2_any,
             wout2_any, out_ref, stage, rs_buf, ag_buf, ag_src,
             win1_ref, wout1_ref, win2_ref, wout2_ref,
             rs_send, rs_recv, ag_send, ag_recv, wsem):
        my = lax.axis_index("i")
        pending = []

        wcopies = []
        for i, (src, dst) in enumerate([
            (win1_any, win1_ref), (wout1_any, wout1_ref),
            (win2_any, win2_ref), (wout2_any, wout2_ref),
        ]):
            c = pltpu.make_async_copy(src, dst, wsem.at[i])
            c.start()
            wcopies.append(c)

        barrier_sem = pltpu.get_barrier_semaphore()
        for k in range(1, N_DEV):
            pl.semaphore_signal(
                barrier_sem, inc=1,
                device_id=(lax.rem(my + k, N_DEV),),
                device_id_type=pl.DeviceIdType.MESH,
            )
        pl.semaphore_wait(barrier_sem, N_DEV - 1)

        def mlp(xv, win_ref, wout_ref):
            hv = jnp.maximum(
                jnp.dot(xv, win_ref[...], preferred_element_type=jnp.float32),
                0.0,
            )
            return jnp.dot(hv, wout_ref[...],
                           preferred_element_type=jnp.float32)

        def rs_send_block(bnd, k, val):
            stage[bnd, k - 1] = val.astype(jnp.bfloat16)
            dst = lax.rem(my + k, N_DEV)
            r = pltpu.make_async_remote_copy(
                src_ref=stage.at[bnd, k - 1],
                dst_ref=rs_buf.at[bnd, k],
                send_sem=rs_send.at[bnd, k],
                recv_sem=rs_recv.at[bnd, k],
                device_id=(dst,),
                device_id_type=pl.DeviceIdType.MESH,
            )
            r.start()
            pending.append(r)
            return r

        rs0 = []
        for k in (2, 1, 3):
            dst = lax.rem(my + k, N_DEV)
            xb = x_ref[pl.ds(dst * rb, rb), :]
            rs0.append(rs_send_block(0, k, mlp(xb, win0_ref, wout0_ref)))
        p_local = mlp(x_ref[pl.ds(my * rb, rb), :], win0_ref, wout0_ref)

        for bnd, (win_ref, wout_ref) in enumerate(
            [(win1_ref, wout1_ref), (win2_ref, wout2_ref)]
        ):
            rs_prev = rs0 if bnd == 0 else rs_next
            wcopies[2 * bnd].wait()
            wcopies[2 * bnd + 1].wait()
            for r in rs_prev:
                r.wait_recv()
            x_own = p_local + (rs_buf[bnd, 1].astype(jnp.float32)
                               + rs_buf[bnd, 2].astype(jnp.float32)
                               + rs_buf[bnd, 3].astype(jnp.float32))
            ag_src[bnd] = x_own.astype(jnp.bfloat16)
            ags = []
            for k in range(1, N_DEV):
                dst = lax.rem(my + k, N_DEV)
                r = pltpu.make_async_remote_copy(
                    src_ref=ag_src.at[bnd],
                    dst_ref=ag_buf.at[bnd, k],
                    send_sem=ag_send.at[bnd, k],
                    recv_sem=ag_recv.at[bnd, k],
                    device_id=(dst,),
                    device_id_type=pl.DeviceIdType.MESH,
                )
                r.start()
                pending.append(r)
                ags.append(r)
            rs_next = []
            rs_next.append(
                rs_send_block(bnd + 1, 2, mlp(x_own, win_ref, wout_ref)))
            for k in (1, 3):
                ags[k - 1].wait_recv()
                blk = mlp(ag_buf[bnd, k].astype(jnp.float32),
                          win_ref, wout_ref)
                rs_next.append(rs_send_block(bnd + 1, k, blk))
            ags[1].wait_recv()
            p_local = mlp(ag_buf[bnd, 2].astype(jnp.float32),
                          win_ref, wout_ref)

        for r in rs_next:
            r.wait_recv()
        out_ref[...] = p_local + (rs_buf[2, 1].astype(jnp.float32)
                                  + rs_buf[2, 2].astype(jnp.float32)
                                  + rs_buf[2, 3].astype(jnp.float32))

        for r in pending:
            r.wait_send()

    return pl.pallas_call(
        body,
        out_shape=jax.ShapeDtypeStruct((rb, d), jnp.float32),
        in_specs=(
            [pl.BlockSpec(memory_space=pltpu.VMEM)] * 3
            + [pl.BlockSpec(memory_space=pltpu.MemorySpace.HBM)] * 4
        ),
        out_specs=pl.BlockSpec(memory_space=pltpu.VMEM),
        scratch_shapes=[
            pltpu.VMEM((3, N_DEV - 1, rb, d), jnp.bfloat16),
            pltpu.VMEM((3, N_DEV, rb, d), jnp.bfloat16),
            pltpu.VMEM((2, N_DEV, rb, d), jnp.bfloat16),
            pltpu.VMEM((2, rb, d), jnp.bfloat16),
            pltpu.VMEM(Win1.shape, jnp.float32),
            pltpu.VMEM(Wout1.shape, jnp.float32),
            pltpu.VMEM(Win2.shape, jnp.float32),
            pltpu.VMEM(Wout2.shape, jnp.float32),
            pltpu.SemaphoreType.DMA((3, N_DEV)),
            pltpu.SemaphoreType.DMA((3, N_DEV)),
            pltpu.SemaphoreType.DMA((2, N_DEV)),
            pltpu.SemaphoreType.DMA((2, N_DEV)),
            pltpu.SemaphoreType.DMA((4,)),
        ],
        compiler_params=pltpu.CompilerParams(collective_id=0),
    )(x, Win0, Wout0, Win1, Wout1, Win2, Wout2)


# device time: 23946 ns/iter; 1.0529x vs baseline; 1.0529x over previous
import jax
import jax.numpy as jnp
from jax import lax
from jax.experimental import pallas as pl
from jax.experimental.pallas import tpu as pltpu

N_DEV = 4


def kernel(x, Win0, Wout0, Win1, Wout1, Win2, Wout2):
    b_rows, d = x.shape
    rb = b_rows // N_DEV

    def body(x_ref, win0_ref, wout0_ref, win1_any, wout1_any, win2_any,
             wout2_any, out_ref, stage, rs_buf, ag_buf, ag_src,
             win1_ref, wout1_ref, win2_ref, wout2_ref,
             rs_send, rs_recv, ag_send, ag_recv, wsem):
        my = lax.axis_index("i")
        pending = []

        barrier_sem = pltpu.get_barrier_semaphore()
        for k in range(1, N_DEV):
            pl.semaphore_signal(
                barrier_sem, inc=1,
                device_id=(lax.rem(my + k, N_DEV),),
                device_id_type=pl.DeviceIdType.MESH,
            )

        wcopies = []
        for i, (src, dst) in enumerate([
            (win1_any, win1_ref), (wout1_any, wout1_ref),
            (win2_any, win2_ref), (wout2_any, wout2_ref),
        ]):
            c = pltpu.make_async_copy(src, dst, wsem.at[i])
            c.start()
            wcopies.append(c)

        def mlp(xv, win_ref, wout_ref):
            hv = jnp.maximum(
                jnp.dot(xv, win_ref[...], preferred_element_type=jnp.float32),
                0.0,
            )
            return jnp.dot(hv, wout_ref[...],
                           preferred_element_type=jnp.float32)

        def rs_send_block(bnd, k, val):
            stage[bnd, k - 1] = val.astype(jnp.bfloat16)
            dst = lax.rem(my + k, N_DEV)
            r = pltpu.make_async_remote_copy(
                src_ref=stage.at[bnd, k - 1],
                dst_ref=rs_buf.at[bnd, k],
                send_sem=rs_send.at[bnd, k],
                recv_sem=rs_recv.at[bnd, k],
                device_id=(dst,),
                device_id_type=pl.DeviceIdType.MESH,
            )
            r.start()
            pending.append(r)
            return r

        rs0 = []
        dst2 = lax.rem(my + 2, N_DEV)
        diag_blk = mlp(x_ref[pl.ds(dst2 * rb, rb), :], win0_ref, wout0_ref)
        pl.semaphore_wait(barrier_sem, N_DEV - 1)
        rs0.append(rs_send_block(0, 2, diag_blk))
        dst1 = lax.rem(my + 1, N_DEV)
        dst3 = lax.rem(my + 3, N_DEV)
        nb = mlp(jnp.concatenate(
            [x_ref[pl.ds(dst1 * rb, rb), :], x_ref[pl.ds(dst3 * rb, rb), :]],
            axis=0), win0_ref, wout0_ref)
        rs0.append(rs_send_block(0, 1, nb[:rb, :]))
        rs0.append(rs_send_block(0, 3, nb[rb:, :]))
        p_local = mlp(x_ref[pl.ds(my * rb, rb), :], win0_ref, wout0_ref)

        for bnd, (win_ref, wout_ref) in enumerate(
            [(win1_ref, wout1_ref), (win2_ref, wout2_ref)]
        ):
            rs_prev = rs0 if bnd == 0 else rs_next
            wcopies[2 * bnd].wait()
            wcopies[2 * bnd + 1].wait()
            for r in rs_prev:
                r.wait_recv()
            x_own = p_local + (rs_buf[bnd, 1].astype(jnp.float32)
                               + rs_buf[bnd, 2].astype(jnp.float32)
                               + rs_buf[bnd, 3].astype(jnp.float32))
            ag_src[bnd] = x_own.astype(jnp.bfloat16)
            ags = {}
            for k in (1, 3, 2):
                dst = lax.rem(my + k, N_DEV)
                r = pltpu.make_async_remote_copy(
                    src_ref=ag_src.at[bnd],
                    dst_ref=ag_buf.at[bnd, k],
                    send_sem=ag_send.at[bnd, k],
                    recv_sem=ag_recv.at[bnd, k],
                    device_id=(dst,),
                    device_id_type=pl.DeviceIdType.MESH,
                )
                r.start()
                pending.append(r)
                ags[k] = r
            rs_next = []
            rs_next.append(
                rs_send_block(bnd + 1, 2, mlp(x_own, win_ref, wout_ref)))
            ags[1].wait_recv()
            ags[3].wait_recv()
            nb = mlp(jnp.concatenate(
                [ag_buf[bnd, 1].astype(jnp.float32),
                 ag_buf[bnd, 3].astype(jnp.float32)], axis=0),
                win_ref, wout_ref)
            rs_next.append(rs_send_block(bnd + 1, 1, nb[:rb, :]))
            rs_next.append(rs_send_block(bnd + 1, 3, nb[rb:, :]))
            ags[2].wait_recv()
            p_local = mlp(ag_buf[bnd, 2].astype(jnp.float32),
                          win_ref, wout_ref)

        for r in rs_next:
            r.wait_recv()
        out_ref[...] = p_local + (rs_buf[2, 1].astype(jnp.float32)
                                  + rs_buf[2, 2].astype(jnp.float32)
                                  + rs_buf[2, 3].astype(jnp.float32))

        for r in pending:
            r.wait_send()

    return pl.pallas_call(
        body,
        out_shape=jax.ShapeDtypeStruct((rb, d), jnp.float32),
        in_specs=(
            [pl.BlockSpec(memory_space=pltpu.VMEM)] * 3
            + [pl.BlockSpec(memory_space=pltpu.MemorySpace.HBM)] * 4
        ),
        out_specs=pl.BlockSpec(memory_space=pltpu.VMEM),
        scratch_shapes=[
            pltpu.VMEM((3, N_DEV - 1, rb, d), jnp.bfloat16),
            pltpu.VMEM((3, N_DEV, rb, d), jnp.bfloat16),
            pltpu.VMEM((2, N_DEV, rb, d), jnp.bfloat16),
            pltpu.VMEM((2, rb, d), jnp.bfloat16),
            pltpu.VMEM(Win1.shape, jnp.float32),
            pltpu.VMEM(Wout1.shape, jnp.float32),
            pltpu.VMEM(Win2.shape, jnp.float32),
            pltpu.VMEM(Wout2.shape, jnp.float32),
            pltpu.SemaphoreType.DMA((3, N_DEV)),
            pltpu.SemaphoreType.DMA((3, N_DEV)),
            pltpu.SemaphoreType.DMA((2, N_DEV)),
            pltpu.SemaphoreType.DMA((2, N_DEV)),
            pltpu.SemaphoreType.DMA((4,)),
        ],
        compiler_params=pltpu.CompilerParams(collective_id=0),
    )(x, Win0, Wout0, Win1, Wout1, Win2, Wout2)
